# Initial kernel scaffold; baseline (speedup 1.0000x reference)
import jax
import jax.numpy as jnp
from jax.experimental import pallas as pl


def kernel(x, x_src, e_feat, wn_t, bn, we_t, be, wr_t, br):
    raise NotImplementedError("write your pallas kernel here")


# speedup vs baseline: 1.0797x; 1.0797x over previous
import jax
import jax.numpy as jnp
from jax.experimental import pallas as pl
from jax.experimental.pallas import tpu as pltpu

SUBLANE = 8


def _fused_body(x_ref, xs_ref, es_ref, w1_ref, w2_ref, w3_ref, b_ref, o_ref):
    acc = jnp.dot(x_ref[...], w1_ref[...], preferred_element_type=jnp.float32)
    acc += jnp.dot(xs_ref[...], w2_ref[...], preferred_element_type=jnp.float32)
    acc += jnp.dot(es_ref[...], w3_ref[...], preferred_element_type=jnp.float32)
    o_ref[...] = acc + b_ref[...]


def _pick_tile(n, *, max_tile=4096):
    best = None
    t = SUBLANE
    while t <= min(max_tile, n // 2):
        if n % t == 0:
            best = t
        t += SUBLANE
    return best if best is not None else n


def kernel(x, x_src, e_feat, wn_t, bn, we_t, be, wr_t, br):
    n, fn = x.shape
    _, d, fe = e_feat.shape
    m2 = wn_t.shape[1]
    r = wr_t.shape[1]
    hi = jax.lax.Precision.HIGHEST

    wrn, wre = wr_t[:m2], wr_t[m2:]
    inv_dp1 = 1.0 / (d + 1)
    wn_fold = jnp.dot(wn_t, wrn, precision=hi) * inv_dp1
    we_fold = jnp.dot(we_t, wre, precision=hi) * inv_dp1
    w2 = jnp.tile(wn_fold, (d, 1))
    w3 = jnp.tile(we_fold, (d, 1))
    b_all = (jnp.dot(bn.reshape(1, m2), wrn, precision=hi)
             + (d * inv_dp1) * jnp.dot(be.reshape(1, m2), wre, precision=hi)
             + br.reshape(1, r))

    xs = x_src.reshape(n, d * fn)
    es = e_feat.reshape(n, d * fe)

    tn = _pick_tile(n)
    grid = n // tn

    flops = 2 * n * (fn + d * fn + d * fe) * r + 3 * n * r
    bytes_accessed = 4 * (n * (fn + d * fn + d * fe) + n * r
                          + (fn + d * fn + d * fe) * r + r)

    return pl.pallas_call(
        _fused_body,
        out_shape=jax.ShapeDtypeStruct((n, r), jnp.float32),
        grid=(grid,),
        in_specs=[
            pl.BlockSpec((tn, fn), lambda i: (i, 0)),
            pl.BlockSpec((tn, d * fn), lambda i: (i, 0)),
            pl.BlockSpec((tn, d * fe), lambda i: (i, 0)),
            pl.BlockSpec((fn, r), lambda i: (0, 0)),
            pl.BlockSpec((d * fn, r), lambda i: (0, 0)),
            pl.BlockSpec((d * fe, r), lambda i: (0, 0)),
            pl.BlockSpec((1, r), lambda i: (0, 0)),
        ],
        out_specs=pl.BlockSpec((tn, r), lambda i: (i, 0)),
        compiler_params=pltpu.CompilerParams(
            dimension_semantics=("parallel",),
            vmem_limit_bytes=32 * 1024 * 1024),
        cost_estimate=pl.CostEstimate(flops=flops, transcendentals=0,
                                      bytes_accessed=bytes_accessed),
    )(x, xs, es, wn_fold, w2, w3, b_all)


# speedup vs baseline: 9.5183x; 8.8158x over previous
import jax
import jax.numpy as jnp
from jax.experimental import pallas as pl
from jax.experimental.pallas import tpu as pltpu

LANE = 128


def _fused_body(xt_ref, xst_ref, est_ref, w1t_ref, w2t_ref, w3t_ref, bt_ref,
                o_ref):
    acc = jnp.dot(w1t_ref[...], xt_ref[...],
                  preferred_element_type=jnp.float32)
    acc += jnp.dot(w2t_ref[...], xst_ref[...],
                   preferred_element_type=jnp.float32)
    acc += jnp.dot(w3t_ref[...], est_ref[...],
                   preferred_element_type=jnp.float32)
    o_ref[...] = acc + bt_ref[...]


def _pick_lane_tile(n, *, max_tile=8192):
    best = None
    t = LANE
    while t <= min(max_tile, n // 2):
        if n % t == 0:
            best = t
        t += LANE
    return best if best is not None else n


def kernel(x, x_src, e_feat, wn_t, bn, we_t, be, wr_t, br):
    n, fn = x.shape
    _, d, fe = e_feat.shape
    m2 = wn_t.shape[1]
    r = wr_t.shape[1]
    hi = jax.lax.Precision.HIGHEST

    wrn, wre = wr_t[:m2], wr_t[m2:]
    inv_dp1 = 1.0 / (d + 1)
    wn_fold_t = jnp.dot(wrn.T, wn_t.T, precision=hi) * inv_dp1
    we_fold_t = jnp.dot(wre.T, we_t.T, precision=hi) * inv_dp1
    w1t = wn_fold_t
    w2t = jnp.tile(wn_fold_t, (1, d))
    w3t = jnp.repeat(we_fold_t, d, axis=1)
    bt = (jnp.dot(bn.reshape(1, m2), wrn, precision=hi)
          + (d * inv_dp1) * jnp.dot(be.reshape(1, m2), wre, precision=hi)
          + br.reshape(1, r)).reshape(r, 1)

    xt = x.T
    xst = x_src.transpose(1, 2, 0).reshape(d * fn, n)
    est = e_feat.transpose(2, 1, 0).reshape(fe * d, n)

    tn = _pick_lane_tile(n)
    grid = n // tn

    k = fn + d * fn + fe * d
    flops = 2 * n * k * r + n * r
    bytes_accessed = 4 * (n * k + n * r + k * r + r)

    out_t = pl.pallas_call(
        _fused_body,
        out_shape=jax.ShapeDtypeStruct((r, n), jnp.float32),
        grid=(grid,),
        in_specs=[
            pl.BlockSpec((fn, tn), lambda i: (0, i)),
            pl.BlockSpec((d * fn, tn), lambda i: (0, i)),
            pl.BlockSpec((fe * d, tn), lambda i: (0, i)),
            pl.BlockSpec((r, fn), lambda i: (0, 0)),
            pl.BlockSpec((r, d * fn), lambda i: (0, 0)),
            pl.BlockSpec((r, fe * d), lambda i: (0, 0)),
            pl.BlockSpec((r, 1), lambda i: (0, 0)),
        ],
        out_specs=pl.BlockSpec((r, tn), lambda i: (0, i)),
        compiler_params=pltpu.CompilerParams(
            dimension_semantics=("parallel",),
            vmem_limit_bytes=32 * 1024 * 1024),
        cost_estimate=pl.CostEstimate(flops=flops, transcendentals=0,
                                      bytes_accessed=bytes_accessed),
    )(xt, xst, est, w1t, w2t, w3t, bt)
    return out_t.T


# speedup vs baseline: 10.3157x; 1.0838x over previous
import jax
import jax.numpy as jnp
from jax.experimental import pallas as pl
from jax.experimental.pallas import tpu as pltpu

LANE = 128


def _fused_body(xt_ref, xst_ref, est_ref, w1t_ref, w2t_ref, w3t_ref, bt_ref,
                o_ref):
    acc = jnp.dot(w1t_ref[...], xt_ref[...],
                  preferred_element_type=jnp.float32)
    acc += jnp.dot(w2t_ref[...], xst_ref[...],
                   preferred_element_type=jnp.float32)
    acc += jnp.dot(w3t_ref[...], est_ref[...],
                   preferred_element_type=jnp.float32)
    o_ref[...] = acc + bt_ref[...]


def _pick_lane_tile(n, *, max_tile=16384):
    best = None
    t = LANE
    while t <= min(max_tile, n // 2):
        if n % t == 0:
            best = t
        t += LANE
    return best if best is not None else n


def kernel(x, x_src, e_feat, wn_t, bn, we_t, be, wr_t, br):
    n, fn = x.shape
    _, d, fe = e_feat.shape
    m2 = wn_t.shape[1]
    r = wr_t.shape[1]
    hi = jax.lax.Precision.HIGHEST

    wrn, wre = wr_t[:m2], wr_t[m2:]
    inv_dp1 = 1.0 / (d + 1)
    wn_fold_t = jnp.dot(wrn.T, wn_t.T, precision=hi) * inv_dp1
    we_fold_t = jnp.dot(wre.T, we_t.T, precision=hi) * inv_dp1
    w1t = wn_fold_t
    w2t = jnp.tile(wn_fold_t, (1, d))
    w3t = jnp.repeat(we_fold_t, d, axis=1)
    bt = (jnp.dot(bn.reshape(1, m2), wrn, precision=hi)
          + (d * inv_dp1) * jnp.dot(be.reshape(1, m2), wre, precision=hi)
          + br.reshape(1, r)).reshape(r, 1)

    xt = x.T
    xst = x_src.transpose(1, 2, 0).reshape(d * fn, n)
    est = e_feat.transpose(2, 1, 0).reshape(fe * d, n)

    tn = _pick_lane_tile(n)
    grid = n // tn

    k = fn + d * fn + fe * d
    flops = 2 * n * k * r + n * r
    bytes_accessed = 4 * (n * k + n * r + k * r + r)

    out_t = pl.pallas_call(
        _fused_body,
        out_shape=jax.ShapeDtypeStruct((r, n), jnp.float32),
        grid=(grid,),
        in_specs=[
            pl.BlockSpec((fn, tn), lambda i: (0, i)),
            pl.BlockSpec((d * fn, tn), lambda i: (0, i)),
            pl.BlockSpec((fe * d, tn), lambda i: (0, i)),
            pl.BlockSpec((r, fn), lambda i: (0, 0)),
            pl.BlockSpec((r, d * fn), lambda i: (0, 0)),
            pl.BlockSpec((r, fe * d), lambda i: (0, 0)),
            pl.BlockSpec((r, 1), lambda i: (0, 0)),
        ],
        out_specs=pl.BlockSpec((r, tn), lambda i: (0, i)),
        compiler_params=pltpu.CompilerParams(
            dimension_semantics=("parallel",),
            vmem_limit_bytes=32 * 1024 * 1024),
        cost_estimate=pl.CostEstimate(flops=flops, transcendentals=0,
                                      bytes_accessed=bytes_accessed),
    )(xt, xst, est, w1t, w2t, w3t, bt)
    return out_t.T
